# Initial kernel scaffold; baseline (speedup 1.0000x reference)
import jax
import jax.numpy as jnp
from jax.experimental import pallas as pl


def kernel(source_nodes, destination_nodes, negative_nodes, edge_times, edge_idxs, neighbors, neighbor_eidx, neighbor_times, node_feats, edge_feats, time_w, time_b, Wq, bq, Wk, bk, Wv, bv, Wo, bo, fc1_w, fc1_b, fc2_w, fc2_b):
    raise NotImplementedError("write your pallas kernel here")


# speedup vs baseline: 1.1057x; 1.1057x over previous
import functools

import jax
import jax.numpy as jnp
import numpy as np
from jax import lax
from jax.experimental import pallas as pl
from jax.experimental.pallas import tpu as pltpu
from jax.experimental.pallas import tpu_sc as plsc

_B3 = 6000
_K = 20
_DF = 128
_DE = 16
_DT = 128
_QD = 256
_HD = 128
_R_TOT = _B3 * _K

_NC, _NS = 2, 16
_NW = _NC * _NS
_CH = 128

_SRC_CHUNKS = 2
_NEI_CHUNKS = 32
_EDG_CHUNKS = 30
_SRC_PAD = _NW * _SRC_CHUNKS * _CH
_NEI_PAD = _NW * _NEI_CHUNKS * _CH
_EDG_PAD = _NW * _EDG_CHUNKS * _CH


def _sc_gather_body(nf_hbm, ef_hbm, sidx_hbm, nidx_hbm, eidx_hbm,
                    src_out, nei_out, edg_out,
                    sidx_v, nidx_v, eidx_v, rows_v, erows_v, sem):
    wid = lax.axis_index("s") * _NC + lax.axis_index("c")
    pltpu.sync_copy(sidx_hbm.at[wid], sidx_v)
    pltpu.sync_copy(nidx_hbm.at[wid], nidx_v)
    pltpu.sync_copy(eidx_hbm.at[wid], eidx_v)

    sbase = wid * (_SRC_CHUNKS * _CH)

    def sloop(j, c):
        pltpu.async_copy(nf_hbm.at[sidx_v.at[j]], rows_v, sem).wait()
        pltpu.sync_copy(rows_v, src_out.at[pl.ds(sbase + j * _CH, _CH)])
        return c

    lax.fori_loop(0, _SRC_CHUNKS, sloop, 0)

    nbase = wid * (_NEI_CHUNKS * _CH)

    def nloop(j, c):
        pltpu.async_copy(nf_hbm.at[nidx_v.at[j]], rows_v, sem).wait()
        pltpu.sync_copy(rows_v, nei_out.at[pl.ds(nbase + j * _CH, _CH)])
        return c

    lax.fori_loop(0, _NEI_CHUNKS, nloop, 0)

    ebase = wid * (_EDG_CHUNKS * _CH)

    def eloop(j, c):
        pltpu.async_copy(ef_hbm.at[eidx_v.at[j]], erows_v, sem).wait()
        pltpu.sync_copy(erows_v, edg_out.at[pl.ds(ebase + j * _CH, _CH)])
        return c

    lax.fori_loop(0, _EDG_CHUNKS, eloop, 0)


def _sc_gather(node_feats, edge_feats, sidx, nidx, eidx):
    mesh = plsc.VectorSubcoreMesh(
        core_axis_name="c", subcore_axis_name="s", num_cores=_NC)
    fn = pl.kernel(
        _sc_gather_body,
        out_type=(
            jax.ShapeDtypeStruct((_SRC_PAD, _DF), jnp.float32),
            jax.ShapeDtypeStruct((_NEI_PAD, _DF), jnp.float32),
            jax.ShapeDtypeStruct((_EDG_PAD, _DE), jnp.float32),
        ),
        mesh=mesh,
        scratch_types=[
            pltpu.VMEM((_SRC_CHUNKS, _CH), jnp.int32),
            pltpu.VMEM((_NEI_CHUNKS, _CH), jnp.int32),
            pltpu.VMEM((_EDG_CHUNKS, _CH), jnp.int32),
            pltpu.VMEM((_CH, _DF), jnp.float32),
            pltpu.VMEM((_CH, _DE), jnp.float32),
            pltpu.SemaphoreType.DMA,
        ],
        compiler_params=pltpu.CompilerParams(use_tc_tiling_on_sc=False),
    )
    return fn(node_feats, edge_feats, sidx, nidx, eidx)


def _tc_body(src_ref, nei_ref, edg_ref, ts_ref, nt_ref, nbr_ref,
             tw_ref, tb_ref,
             wqn_ref, wqt_ref, bq_ref,
             wkn_ref, wke_ref, wkt_ref, bk_ref,
             wvn_ref, wve_ref, wvt_ref, bv_ref,
             woa_ref, wob_ref, bo_ref,
             f1o_ref, f1s_ref, f1b_ref,
             f2_ref, f2b_ref,
             out_ref):
    nb = src_ref.shape[0]
    r = nb * _K
    f32 = jnp.float32

    nf = nei_ref[...]
    ef = edg_ref[...]
    tw = tw_ref[...]
    tb = tb_ref[...]

    delta = ts_ref[...] - nt_ref[...]
    te3 = jnp.cos(delta[:, :, None] * tw[0][None, None, :]
                  + tb[0][None, None, :])
    te = te3.reshape(r, _DT)

    dot = functools.partial(jnp.dot, preferred_element_type=f32)

    k = (dot(nf, wkn_ref[...]) + dot(ef, wke_ref[...]) + dot(te, wkt_ref[...])
         + bk_ref[...])
    v = (dot(nf, wvn_ref[...]) + dot(ef, wve_ref[...]) + dot(te, wvt_ref[...])
         + bv_ref[...])

    sf = src_ref[...]
    cosb = jnp.cos(tb)
    q = dot(sf, wqn_ref[...]) + dot(cosb, wqt_ref[...]) + bq_ref[...]

    k3 = k.reshape(nb, _K, _QD)
    v3 = v.reshape(nb, _K, _QD)
    invalid = nbr_ref[...] == 0
    scale = f32(1.0 / np.sqrt(_HD))

    outs = []
    for h in range(2):
        qh = q[:, h * _HD:(h + 1) * _HD]
        kh = k3[:, :, h * _HD:(h + 1) * _HD]
        vh = v3[:, :, h * _HD:(h + 1) * _HD]
        s = jnp.sum(kh * qh[:, None, :], axis=-1) * scale
        s = jnp.where(invalid, f32(-1e10), s)
        m = jnp.max(s, axis=-1, keepdims=True)
        e = jnp.exp(s - m)
        a = e / jnp.sum(e, axis=-1, keepdims=True)
        outs.append(jnp.sum(a[:, :, None] * vh, axis=1))

    o = dot(outs[0], woa_ref[...]) + dot(outs[1], wob_ref[...]) + bo_ref[...]
    all_inv = jnp.all(invalid, axis=-1, keepdims=True)
    o = jnp.where(all_inv, f32(0.0), o)

    h1 = jax.nn.relu(dot(o, f1o_ref[...]) + dot(sf, f1s_ref[...]) + f1b_ref[...])
    out_ref[...] = dot(h1, f2_ref[...]) + f2b_ref[...]


def kernel(source_nodes, destination_nodes, negative_nodes, edge_times,
           edge_idxs, neighbors, neighbor_eidx, neighbor_times, node_feats,
           edge_feats, time_w, time_b, Wq, bq, Wk, bk, Wv, bv, Wo, bo,
           fc1_w, fc1_b, fc2_w, fc2_b):
    i32 = jnp.int32
    nodes = jnp.concatenate(
        [source_nodes, destination_nodes, negative_nodes]).astype(i32)
    sidx = jnp.zeros((_SRC_PAD,), i32).at[:_B3].set(nodes)
    nidx = jnp.zeros((_NEI_PAD,), i32).at[:_R_TOT].set(
        neighbors.reshape(-1).astype(i32))
    eidx = jnp.zeros((_EDG_PAD,), i32).at[:_R_TOT].set(
        neighbor_eidx.reshape(-1).astype(i32))
    sidx = sidx.reshape(_NW, _SRC_CHUNKS, _CH)
    nidx = nidx.reshape(_NW, _NEI_CHUNKS, _CH)
    eidx = eidx.reshape(_NW, _EDG_CHUNKS, _CH)

    src_rows, nei_rows, edg_rows = _sc_gather(
        node_feats, edge_feats, sidx, nidx, eidx)

    ts3 = jnp.tile(edge_times, 3).reshape(_B3, 1)

    nb = 200
    grid = _B3 // nb
    rpb = nb * _K

    def ev(i):
        return (i, 0)

    def full(i):
        return (0, 0)

    spec = pl.BlockSpec
    out = pl.pallas_call(
        _tc_body,
        grid=(grid,),
        in_specs=[
            spec((nb, _DF), ev),
            spec((rpb, _DF), ev),
            spec((rpb, _DE), ev),
            spec((nb, 1), ev),
            spec((nb, _K), ev),
            spec((nb, _K), ev),
            spec((1, _DT), full),
            spec((1, _DT), full),
            spec((_DF, _QD), full),
            spec((_DT, _QD), full),
            spec((1, _QD), full),
            spec((_DF, _QD), full),
            spec((_DE, _QD), full),
            spec((_DT, _QD), full),
            spec((1, _QD), full),
            spec((_DF, _QD), full),
            spec((_DE, _QD), full),
            spec((_DT, _QD), full),
            spec((1, _QD), full),
            spec((_HD, _QD), full),
            spec((_HD, _QD), full),
            spec((1, _QD), full),
            spec((_QD, _DF), full),
            spec((_DF, _DF), full),
            spec((1, _DF), full),
            spec((_DF, _DF), full),
            spec((1, _DF), full),
        ],
        out_specs=spec((nb, _DF), ev),
        out_shape=jax.ShapeDtypeStruct((_B3, _DF), jnp.float32),
        compiler_params=pltpu.CompilerParams(
            dimension_semantics=("arbitrary",)),
    )(
        src_rows, nei_rows, edg_rows, ts3,
        neighbor_times, neighbors.astype(i32),
        time_w.reshape(1, _DT), time_b.reshape(1, _DT),
        Wq[:_DF], Wq[_DF:], bq.reshape(1, _QD),
        Wk[:_DF], Wk[_DF:_DF + _DE], Wk[_DF + _DE:], bk.reshape(1, _QD),
        Wv[:_DF], Wv[_DF:_DF + _DE], Wv[_DF + _DE:], bv.reshape(1, _QD),
        Wo[:_HD], Wo[_HD:], bo.reshape(1, _QD),
        fc1_w[:_QD], fc1_w[_QD:], fc1_b.reshape(1, _DF),
        fc2_w, fc2_b.reshape(1, _DF),
    )
    return out
